# baseline (speedup 1.0000x reference)
import jax
import jax.numpy as jnp
from jax import lax
from jax.experimental import pallas as pl
from jax.experimental.pallas import tpu as pltpu
from jax.experimental.pallas import tpu_sc as plsc

BSZ = 4
SEQ = 4096
D_MODEL = 1024
NC = 2
NS = 16
NW = NC * NS
ROWS_PER_W = SEQ // NW
CHUNK = 64
NCHUNK = ROWS_PER_W // CHUNK

TC_CHUNK = 512
TC_NCHUNK = SEQ // TC_CHUNK
TC_NBUF = 2


def _sc_gather_body(pe_hbm, out_hbm, buf0, buf1, sem_r0, sem_r1, sem_w):
    wid = lax.axis_index("s") * NC + lax.axis_index("c")
    base = wid * ROWS_PER_W
    r0 = pltpu.async_copy(pe_hbm.at[pl.ds(base, CHUNK)], buf0, sem_r0)
    r1 = pltpu.async_copy(pe_hbm.at[pl.ds(base + CHUNK, CHUNK)], buf1, sem_r1)
    r0.wait()
    w0 = pltpu.async_copy(buf0, out_hbm.at[0, pl.ds(base, CHUNK)], sem_w)
    r1.wait()
    w1 = pltpu.async_copy(buf1, out_hbm.at[0, pl.ds(base + CHUNK, CHUNK)], sem_w)
    w0.wait()
    w1.wait()


def _sc_gather(pe):
    mesh = plsc.VectorSubcoreMesh(core_axis_name="c", subcore_axis_name="s",
                                  num_cores=NC, num_subcores=NS)
    f = pl.kernel(
        _sc_gather_body,
        mesh=mesh,
        out_type=jax.ShapeDtypeStruct((BSZ, SEQ, D_MODEL), jnp.float32),
        scratch_types=[
            pltpu.VMEM((CHUNK, D_MODEL), jnp.float32),
            pltpu.VMEM((CHUNK, D_MODEL), jnp.float32),
            pltpu.SemaphoreType.DMA,
            pltpu.SemaphoreType.DMA,
            pltpu.SemaphoreType.DMA,
        ],
    )
    return f(pe)


def _tc_broadcast_body(aliased_ref, pe_hbm, out_ref, buf0, buf1, sem_r, sem_w):
    del aliased_ref
    bufs = (buf0, buf1)
    reads = [None] * TC_NCHUNK
    writes = [[] for _ in range(TC_NCHUNK)]
    for i in range(TC_NBUF):
        reads[i] = pltpu.async_copy(
            pe_hbm.at[pl.ds(i * TC_CHUNK, TC_CHUNK)], bufs[i], sem_r)
    for i in range(TC_NCHUNK):
        buf = bufs[i % TC_NBUF]
        reads[i].wait()
        for b in range(1, BSZ):
            writes[i].append(pltpu.async_copy(
                buf, out_ref.at[b, pl.ds(i * TC_CHUNK, TC_CHUNK)], sem_w))
        nxt = i + TC_NBUF
        if nxt < TC_NCHUNK:
            for w in writes[i]:
                w.wait()
            reads[nxt] = pltpu.async_copy(
                pe_hbm.at[pl.ds(nxt * TC_CHUNK, TC_CHUNK)],
                bufs[nxt % TC_NBUF], sem_r)
    for i in range(max(TC_NCHUNK - TC_NBUF, 0), TC_NCHUNK):
        for w in writes[i]:
            w.wait()


def _tc_broadcast(partial_out, pe):
    return pl.pallas_call(
        _tc_broadcast_body,
        in_specs=[
            pl.BlockSpec(memory_space=pl.ANY),
            pl.BlockSpec(memory_space=pl.ANY),
        ],
        out_specs=pl.BlockSpec(memory_space=pl.ANY),
        out_shape=jax.ShapeDtypeStruct((BSZ, SEQ, D_MODEL), jnp.float32),
        scratch_shapes=[
            pltpu.VMEM((TC_CHUNK, D_MODEL), jnp.float32),
            pltpu.VMEM((TC_CHUNK, D_MODEL), jnp.float32),
            pltpu.SemaphoreType.DMA,
            pltpu.SemaphoreType.DMA,
        ],
        input_output_aliases={0: 0},
    )(partial_out, pe)


@jax.jit
def _pe_broadcast(pe):
    partial_out = _sc_gather(pe)
    return _tc_broadcast(partial_out, pe)


def kernel(input, pe):
    del input
    return _pe_broadcast(pe)


# speedup vs baseline: 1.2464x; 1.2464x over previous
import jax
import jax.numpy as jnp
from jax import lax
from jax.experimental import pallas as pl
from jax.experimental.pallas import tpu as pltpu
from jax.experimental.pallas import tpu_sc as plsc

BSZ = 4
SEQ = 4096
D_MODEL = 1024
NC = 2
NS = 16
NW = NC * NS
ROWS_PER_W = SEQ // NW
CHUNK = 64


def _pe_broadcast_body(pe_hbm, out_hbm, buf0, buf1, sem_r0, sem_r1, sem_w):
    wid = lax.axis_index("s") * NC + lax.axis_index("c")
    base = wid * ROWS_PER_W
    r0 = pltpu.async_copy(pe_hbm.at[pl.ds(base, CHUNK)], buf0, sem_r0)
    r1 = pltpu.async_copy(pe_hbm.at[pl.ds(base + CHUNK, CHUNK)], buf1, sem_r1)
    writes = []
    r0.wait()
    for b in range(BSZ):
        writes.append(pltpu.async_copy(buf0, out_hbm.at[b, pl.ds(base, CHUNK)], sem_w))
    r1.wait()
    for b in range(BSZ):
        writes.append(pltpu.async_copy(buf1, out_hbm.at[b, pl.ds(base + CHUNK, CHUNK)], sem_w))
    for w in writes:
        w.wait()


@jax.jit
def _pe_broadcast(pe):
    mesh = plsc.VectorSubcoreMesh(core_axis_name="c", subcore_axis_name="s",
                                  num_cores=NC, num_subcores=NS)
    f = pl.kernel(
        _pe_broadcast_body,
        mesh=mesh,
        out_type=jax.ShapeDtypeStruct((BSZ, SEQ, D_MODEL), jnp.float32),
        scratch_types=[
            pltpu.VMEM((CHUNK, D_MODEL), jnp.float32),
            pltpu.VMEM((CHUNK, D_MODEL), jnp.float32),
            pltpu.SemaphoreType.DMA,
            pltpu.SemaphoreType.DMA,
            pltpu.SemaphoreType.DMA,
        ],
    )
    return f(pe)


def kernel(input, pe):
    del input
    return _pe_broadcast(pe)
